# baseline (speedup 1.0000x reference)
import jax
import jax.numpy as jnp
from jax.experimental import pallas as pl
from jax.experimental.pallas import tpu as pltpu


def _vq_block_kernel(x_ref, cb_ref, q_ref, idx_ref, loss_ref):
    i = pl.program_id(0)
    x = x_ref[...]
    cb = cb_ref[...]

    xs = jnp.sum(x * x, axis=1, keepdims=True)
    xn = x / jnp.maximum(jnp.sqrt(xs), 1e-12)
    cs = jnp.sum(cb * cb, axis=1, keepdims=True)
    en = cb / jnp.maximum(jnp.sqrt(cs), 1e-12)

    x2 = jnp.sum(xn * xn, axis=1, keepdims=True)
    e2 = jax.lax.dot_general(
        jnp.ones((1, en.shape[1]), jnp.float32), en * en,
        (((1,), (1,)), ((), ())),
        precision=jax.lax.Precision.HIGHEST,
        preferred_element_type=jnp.float32)
    dot = jax.lax.dot_general(
        xn, en, (((1,), (1,)), ((), ())),
        preferred_element_type=jnp.float32)
    d2 = x2 + e2 - 2.0 * dot
    d2 = jnp.maximum(d2, 0.0)

    idx = jnp.argmin(d2, axis=1).astype(jnp.int32)

    r, k = d2.shape
    onehot = (jax.lax.broadcasted_iota(jnp.int32, (r, k), 1)
              == idx.reshape(r, 1)).astype(jnp.float32)
    q = jax.lax.dot_general(
        onehot, cb, (((1,), (0,)), ((), ())),
        precision=jax.lax.Precision.HIGHEST,
        preferred_element_type=jnp.float32)

    diff = q - x
    q_ref[...] = x + diff
    idx_ref[0, 0, :] = idx
    psum = jnp.sum(diff * diff).reshape(1, 1)

    @pl.when(i == 0)
    def _init():
        loss_ref[...] = jnp.zeros((1, 1), jnp.float32)

    loss_ref[...] += psum


def kernel(x, codebook):
    b, t, d = x.shape
    k = codebook.shape[0]
    n = b * t
    blk = 512
    nb = n // blk
    xf = x.reshape(n, d)

    q, idx, loss_sum = pl.pallas_call(
        _vq_block_kernel,
        grid=(nb,),
        in_specs=[
            pl.BlockSpec((blk, d), lambda i: (i, 0)),
            pl.BlockSpec((k, d), lambda i: (0, 0)),
        ],
        out_specs=[
            pl.BlockSpec((blk, d), lambda i: (i, 0)),
            pl.BlockSpec((1, 1, blk), lambda i: (i, 0, 0)),
            pl.BlockSpec((1, 1), lambda i: (0, 0)),
        ],
        out_shape=[
            jax.ShapeDtypeStruct((n, d), jnp.float32),
            jax.ShapeDtypeStruct((nb, 1, blk), jnp.int32),
            jax.ShapeDtypeStruct((1, 1), jnp.float32),
        ],
    )(xf, codebook)

    quantized_st = q.reshape(b, t, d)
    indices = idx.reshape(b, t)
    quantize_loss = (1.25 / (n * d)) * loss_sum[0, 0]
    return (quantized_st, indices, quantize_loss)


# speedup vs baseline: 1.7092x; 1.7092x over previous
import jax
import jax.numpy as jnp
from jax.experimental import pallas as pl
from jax.experimental.pallas import tpu as pltpu


def _vq_block_kernel(x_ref, cb_ref, q_ref, idx_ref, loss_ref,
                     en_ref, e2_ref, cbb_ref):
    i = pl.program_id(0)

    @pl.when(i == 0)
    def _prep():
        cb = cb_ref[...]
        cs = jnp.sum(cb * cb, axis=1, keepdims=True)
        en = cb / jnp.maximum(jnp.sqrt(cs), 1e-12)
        en_ref[...] = en
        e2_ref[...] = jax.lax.dot_general(
            jnp.ones((1, en.shape[1]), jnp.float32), en * en,
            (((1,), (1,)), ((), ())),
            precision=jax.lax.Precision.HIGHEST,
            preferred_element_type=jnp.float32)
        cbb_ref[...] = cb.astype(jnp.bfloat16)

    x = x_ref[...]
    xs = jnp.sum(x * x, axis=1, keepdims=True)
    xn = x / jnp.maximum(jnp.sqrt(xs), 1e-12)
    x2 = jnp.sum(xn * xn, axis=1, keepdims=True)

    dot = jax.lax.dot_general(
        xn, en_ref[...], (((1,), (1,)), ((), ())),
        preferred_element_type=jnp.float32)
    d2 = x2 + e2_ref[...] - 2.0 * dot
    d2 = jnp.maximum(d2, 0.0)

    idx = jnp.argmin(d2, axis=1).astype(jnp.int32)

    r, k = d2.shape
    onehot = jnp.where(
        jax.lax.broadcasted_iota(jnp.int32, (r, k), 1) == idx.reshape(r, 1),
        jnp.float32(1), jnp.float32(0)).astype(jnp.bfloat16)
    q = jax.lax.dot_general(
        onehot, cbb_ref[...], (((1,), (0,)), ((), ())),
        preferred_element_type=jnp.float32)

    diff = q - x
    q_ref[...] = x + diff
    idx_ref[0, 0, :] = idx
    psum = jnp.sum(diff * diff).reshape(1, 1)

    @pl.when(i == 0)
    def _init():
        loss_ref[...] = jnp.zeros((1, 1), jnp.float32)

    loss_ref[...] += psum


def kernel(x, codebook):
    b, t, d = x.shape
    k = codebook.shape[0]
    n = b * t
    blk = 512
    nb = n // blk
    xf = x.reshape(n, d)

    q, idx, loss_sum = pl.pallas_call(
        _vq_block_kernel,
        grid=(nb,),
        in_specs=[
            pl.BlockSpec((blk, d), lambda i: (i, 0)),
            pl.BlockSpec((k, d), lambda i: (0, 0)),
        ],
        out_specs=[
            pl.BlockSpec((blk, d), lambda i: (i, 0)),
            pl.BlockSpec((1, 1, blk), lambda i: (i, 0, 0)),
            pl.BlockSpec((1, 1), lambda i: (0, 0)),
        ],
        out_shape=[
            jax.ShapeDtypeStruct((n, d), jnp.float32),
            jax.ShapeDtypeStruct((nb, 1, blk), jnp.int32),
            jax.ShapeDtypeStruct((1, 1), jnp.float32),
        ],
        scratch_shapes=[
            pltpu.VMEM((k, d), jnp.float32),
            pltpu.VMEM((1, k), jnp.float32),
            pltpu.VMEM((k, d), jnp.bfloat16),
        ],
    )(xf, codebook)

    quantized_st = q.reshape(b, t, d)
    indices = idx.reshape(b, t)
    quantize_loss = (1.25 / (n * d)) * loss_sum[0, 0]
    return (quantized_st, indices, quantize_loss)


# speedup vs baseline: 2.0346x; 1.1904x over previous
import jax
import jax.numpy as jnp
from jax.experimental import pallas as pl
from jax.experimental.pallas import tpu as pltpu


def _vq_block_kernel(x_ref, cb_ref, q_ref, idx_ref, loss_ref,
                     en_ref, e2_ref, cbb_ref):
    i = pl.program_id(0)

    @pl.when(i == 0)
    def _prep():
        cb = cb_ref[...]
        cs = jnp.sum(cb * cb, axis=1, keepdims=True)
        en = cb / jnp.maximum(jnp.sqrt(cs), 1e-12)
        en_ref[...] = en
        e2_ref[...] = jax.lax.dot_general(
            jnp.ones((1, en.shape[1]), jnp.float32), en * en,
            (((1,), (1,)), ((), ())),
            precision=jax.lax.Precision.HIGHEST,
            preferred_element_type=jnp.float32)
        cbb_ref[...] = cb.astype(jnp.bfloat16)

    x = x_ref[...]
    xs = jnp.sum(x * x, axis=1, keepdims=True)
    xn = x / jnp.maximum(jnp.sqrt(xs), 1e-12)
    x2 = jnp.sum(xn * xn, axis=1, keepdims=True)

    dot = jax.lax.dot_general(
        xn, en_ref[...], (((1,), (1,)), ((), ())),
        preferred_element_type=jnp.float32)
    d2 = x2 + e2_ref[...] - 2.0 * dot

    idx = jnp.argmin(d2, axis=1).astype(jnp.int32)

    r, k = d2.shape
    onehot = jnp.where(
        jax.lax.broadcasted_iota(jnp.int32, (r, k), 1) == idx.reshape(r, 1),
        jnp.float32(1), jnp.float32(0)).astype(jnp.bfloat16)
    q = jax.lax.dot_general(
        onehot, cbb_ref[...], (((1,), (0,)), ((), ())),
        preferred_element_type=jnp.float32)

    diff = q - x
    q_ref[...] = x + diff
    idx_ref[0, 0, :] = idx
    psum = jnp.sum(diff * diff).reshape(1, 1)

    @pl.when(i == 0)
    def _init():
        loss_ref[...] = jnp.zeros((1, 1), jnp.float32)

    loss_ref[...] += psum


def kernel(x, codebook):
    b, t, d = x.shape
    k = codebook.shape[0]
    n = b * t
    blk = 1024
    nb = n // blk
    xf = x.reshape(n, d)

    q, idx, loss_sum = pl.pallas_call(
        _vq_block_kernel,
        grid=(nb,),
        in_specs=[
            pl.BlockSpec((blk, d), lambda i: (i, 0)),
            pl.BlockSpec((k, d), lambda i: (0, 0)),
        ],
        out_specs=[
            pl.BlockSpec((blk, d), lambda i: (i, 0)),
            pl.BlockSpec((1, 1, blk), lambda i: (i, 0, 0)),
            pl.BlockSpec((1, 1), lambda i: (0, 0)),
        ],
        out_shape=[
            jax.ShapeDtypeStruct((n, d), jnp.float32),
            jax.ShapeDtypeStruct((nb, 1, blk), jnp.int32),
            jax.ShapeDtypeStruct((1, 1), jnp.float32),
        ],
        scratch_shapes=[
            pltpu.VMEM((k, d), jnp.float32),
            pltpu.VMEM((1, k), jnp.float32),
            pltpu.VMEM((k, d), jnp.bfloat16),
        ],
    )(xf, codebook)

    quantized_st = q.reshape(b, t, d)
    indices = idx.reshape(b, t)
    quantize_loss = (1.25 / (n * d)) * loss_sum[0, 0]
    return (quantized_st, indices, quantize_loss)


# speedup vs baseline: 2.0641x; 1.0145x over previous
import jax
import jax.numpy as jnp
from jax.experimental import pallas as pl
from jax.experimental.pallas import tpu as pltpu


def _vq_block_kernel(x_ref, cb_ref, q_ref, idx_ref, loss_ref,
                     en_ref, e2_ref, cbb_ref):
    i = pl.program_id(0)

    @pl.when(i == 0)
    def _prep():
        cb = cb_ref[...]
        cs = jnp.sum(cb * cb, axis=1, keepdims=True)
        en = cb / jnp.maximum(jnp.sqrt(cs), 1e-12)
        en_ref[...] = en
        e2_ref[...] = jax.lax.dot_general(
            jnp.ones((1, en.shape[1]), jnp.float32), en * en,
            (((1,), (1,)), ((), ())),
            precision=jax.lax.Precision.HIGHEST,
            preferred_element_type=jnp.float32)
        cbb_ref[...] = cb.astype(jnp.bfloat16)

    x = x_ref[...]
    xs = jnp.sum(x * x, axis=1, keepdims=True)
    xn = x / jnp.maximum(jnp.sqrt(xs), 1e-12)

    dot = jax.lax.dot_general(
        xn, en_ref[...], (((1,), (1,)), ((), ())),
        preferred_element_type=jnp.float32)
    idx = jnp.argmax(dot, axis=1).astype(jnp.int32)

    r, k = dot.shape
    onehot = jnp.where(
        jax.lax.broadcasted_iota(jnp.int32, (r, k), 1) == idx.reshape(r, 1),
        jnp.float32(1), jnp.float32(0)).astype(jnp.bfloat16)
    q = jax.lax.dot_general(
        onehot, cbb_ref[...], (((1,), (0,)), ((), ())),
        preferred_element_type=jnp.float32)

    diff = q - x
    q_ref[...] = x + diff
    idx_ref[0, 0, :] = idx
    psum = jnp.sum(diff * diff).reshape(1, 1)

    @pl.when(i == 0)
    def _init():
        loss_ref[...] = jnp.zeros((1, 1), jnp.float32)

    loss_ref[...] += psum


def kernel(x, codebook):
    b, t, d = x.shape
    k = codebook.shape[0]
    n = b * t
    blk = 1024
    nb = n // blk
    xf = x.reshape(n, d)

    q, idx, loss_sum = pl.pallas_call(
        _vq_block_kernel,
        grid=(nb,),
        in_specs=[
            pl.BlockSpec((blk, d), lambda i: (i, 0)),
            pl.BlockSpec((k, d), lambda i: (0, 0)),
        ],
        out_specs=[
            pl.BlockSpec((blk, d), lambda i: (i, 0)),
            pl.BlockSpec((1, 1, blk), lambda i: (i, 0, 0)),
            pl.BlockSpec((1, 1), lambda i: (0, 0)),
        ],
        out_shape=[
            jax.ShapeDtypeStruct((n, d), jnp.float32),
            jax.ShapeDtypeStruct((nb, 1, blk), jnp.int32),
            jax.ShapeDtypeStruct((1, 1), jnp.float32),
        ],
        scratch_shapes=[
            pltpu.VMEM((k, d), jnp.float32),
            pltpu.VMEM((1, k), jnp.float32),
            pltpu.VMEM((k, d), jnp.bfloat16),
        ],
    )(xf, codebook)

    quantized_st = q.reshape(b, t, d)
    indices = idx.reshape(b, t)
    quantize_loss = (1.25 / (n * d)) * loss_sum[0, 0]
    return (quantized_st, indices, quantize_loss)


# speedup vs baseline: 2.0718x; 1.0037x over previous
import jax
import jax.numpy as jnp
from jax.experimental import pallas as pl
from jax.experimental.pallas import tpu as pltpu


def _vq_block_kernel(x_ref, cb_ref, q_ref, idx_ref, loss_ref,
                     en_ref, cbb_ref):
    i = pl.program_id(0)

    @pl.when(i == 0)
    def _prep():
        cb = cb_ref[...]
        cs = jnp.sum(cb * cb, axis=1, keepdims=True)
        en = cb / jnp.maximum(jnp.sqrt(cs), 1e-12)
        en_ref[...] = en
        cbb_ref[...] = cb.astype(jnp.bfloat16)

    x = x_ref[...]
    xs = jnp.sum(x * x, axis=1, keepdims=True)
    xn = x / jnp.maximum(jnp.sqrt(xs), 1e-12)

    dot = jax.lax.dot_general(
        xn, en_ref[...], (((1,), (1,)), ((), ())),
        preferred_element_type=jnp.float32)
    idx = jnp.argmax(dot, axis=1).astype(jnp.int32)

    r, k = dot.shape
    onehot = jnp.where(
        jax.lax.broadcasted_iota(jnp.int16, (r, k), 1)
        == idx.astype(jnp.int16).reshape(r, 1),
        jnp.bfloat16(1), jnp.bfloat16(0))
    q = jax.lax.dot_general(
        onehot, cbb_ref[...], (((1,), (0,)), ((), ())),
        preferred_element_type=jnp.float32)

    diff = q - x
    q_ref[...] = x + diff
    idx_ref[0, 0, :] = idx
    psum = jnp.sum(diff * diff).reshape(1, 1)

    @pl.when(i == 0)
    def _init():
        loss_ref[...] = jnp.zeros((1, 1), jnp.float32)

    loss_ref[...] += psum


def kernel(x, codebook):
    b, t, d = x.shape
    k = codebook.shape[0]
    n = b * t
    blk = 1024
    nb = n // blk
    xf = x.reshape(n, d)

    q, idx, loss_sum = pl.pallas_call(
        _vq_block_kernel,
        grid=(nb,),
        in_specs=[
            pl.BlockSpec((blk, d), lambda i: (i, 0)),
            pl.BlockSpec((k, d), lambda i: (0, 0)),
        ],
        out_specs=[
            pl.BlockSpec((blk, d), lambda i: (i, 0)),
            pl.BlockSpec((1, 1, blk), lambda i: (i, 0, 0)),
            pl.BlockSpec((1, 1), lambda i: (0, 0)),
        ],
        out_shape=[
            jax.ShapeDtypeStruct((n, d), jnp.float32),
            jax.ShapeDtypeStruct((nb, 1, blk), jnp.int32),
            jax.ShapeDtypeStruct((1, 1), jnp.float32),
        ],
        scratch_shapes=[
            pltpu.VMEM((k, d), jnp.float32),
            pltpu.VMEM((k, d), jnp.bfloat16),
        ],
    )(xf, codebook)

    quantized_st = q.reshape(b, t, d)
    indices = idx.reshape(b, t)
    quantize_loss = (1.25 / (n * d)) * loss_sum[0, 0]
    return (quantized_st, indices, quantize_loss)


# speedup vs baseline: 2.3904x; 1.1537x over previous
import jax
import jax.numpy as jnp
from jax import lax
from jax.experimental import pallas as pl
from jax.experimental.pallas import tpu as pltpu
import jax.experimental.pallas.tpu_sc as plsc

_NUM_CORES = 2
_NUM_SUBCORES = 16
_SUB = 1024


def _tc_argmax_kernel(x_ref, cb_ref, idx_ref, en_ref):
    i = pl.program_id(0)

    @pl.when(i == 0)
    def _prep():
        cb = cb_ref[...]
        cs = jnp.sum(cb * cb, axis=1, keepdims=True)
        en_ref[...] = cb / jnp.maximum(jnp.sqrt(cs), 1e-12)

    x = x_ref[...]
    xs = jnp.sum(x * x, axis=1, keepdims=True)
    xn = x / jnp.maximum(jnp.sqrt(xs), 1e-12)

    dot = jax.lax.dot_general(
        xn, en_ref[...], (((1,), (1,)), ((), ())),
        preferred_element_type=jnp.float32)
    idx_ref[0, 0, :] = jnp.argmax(dot, axis=1).astype(jnp.int32)


def _sc_gather_loss_kernel(idx_hbm, cb_hbm, x_hbm, q_hbm, part_hbm,
                           idx_v, rows_v, x_v, acc_v,
                           sem_i, sem_g, sem_x, sem_o, sem_p):
    c = lax.axis_index("c")
    s = lax.axis_index("s")
    tile = c * _NUM_SUBCORES + s
    n_tiles = _NUM_CORES * _NUM_SUBCORES
    n = q_hbm.shape[0]
    per_tile = n // n_tiles
    n_sub = per_tile // _SUB

    acc_v[...] = jnp.zeros((16,), jnp.float32)

    def sub_chunk(j, _):
        base = tile * per_tile + j * _SUB
        cp_i = pltpu.make_async_copy(
            idx_hbm.at[pl.ds(base, _SUB)], idx_v, sem_i)
        cp_i.start()
        cp_x = pltpu.make_async_copy(
            x_hbm.at[pl.ds(base, _SUB)], x_v, sem_x)
        cp_x.start()
        cp_i.wait()
        cp_g = pltpu.make_async_copy(cb_hbm.at[idx_v], rows_v, sem_g)
        cp_g.start()
        cp_g.wait()
        cp_o = pltpu.make_async_copy(
            rows_v, q_hbm.at[pl.ds(base, _SUB)], sem_o)
        cp_o.start()
        cp_x.wait()

        def tok(i, accs):
            a0, a1 = accs
            d0 = rows_v[i, pl.ds(0, 16)] - x_v[i, pl.ds(0, 16)]
            d1 = rows_v[i, pl.ds(16, 16)] - x_v[i, pl.ds(16, 16)]
            return a0 + d0 * d0, a1 + d1 * d1

        z = jnp.zeros((16,), jnp.float32)
        a0, a1 = lax.fori_loop(0, _SUB, tok, (z, z))
        acc_v[...] += a0 + a1
        cp_o.wait()
        return 0

    lax.fori_loop(0, n_sub, sub_chunk, 0)

    cp_p = pltpu.make_async_copy(acc_v, part_hbm.at[tile], sem_p)
    cp_p.start()
    cp_p.wait()


def kernel(x, codebook):
    b, t, d = x.shape
    k = codebook.shape[0]
    n = b * t
    blk = 1024
    nb = n // blk
    xf = x.reshape(n, d)

    idx = pl.pallas_call(
        _tc_argmax_kernel,
        grid=(nb,),
        in_specs=[
            pl.BlockSpec((blk, d), lambda i: (i, 0)),
            pl.BlockSpec((k, d), lambda i: (0, 0)),
        ],
        out_specs=pl.BlockSpec((1, 1, blk), lambda i: (i, 0, 0)),
        out_shape=jax.ShapeDtypeStruct((nb, 1, blk), jnp.int32),
        scratch_shapes=[pltpu.VMEM((k, d), jnp.float32)],
    )(xf, codebook)
    idx_flat = idx.reshape(n)

    n_tiles = _NUM_CORES * _NUM_SUBCORES
    sc_gather = pl.kernel(
        _sc_gather_loss_kernel,
        out_type=[
            jax.ShapeDtypeStruct((n, d), jnp.float32),
            jax.ShapeDtypeStruct((n_tiles, 16), jnp.float32),
        ],
        mesh=plsc.VectorSubcoreMesh(
            core_axis_name="c", subcore_axis_name="s",
            num_cores=_NUM_CORES, num_subcores=_NUM_SUBCORES),
        compiler_params=pltpu.CompilerParams(use_tc_tiling_on_sc=False),
        scratch_types=[
            pltpu.VMEM((_SUB,), jnp.int32),
            pltpu.VMEM((_SUB, d), jnp.float32),
            pltpu.VMEM((_SUB, d), jnp.float32),
            pltpu.VMEM((16,), jnp.float32),
            pltpu.SemaphoreType.DMA,
            pltpu.SemaphoreType.DMA,
            pltpu.SemaphoreType.DMA,
            pltpu.SemaphoreType.DMA,
            pltpu.SemaphoreType.DMA,
        ],
    )
    q, parts = sc_gather(idx_flat, codebook, xf)

    quantized_st = q.reshape(b, t, d)
    indices = idx.reshape(b, t)
    quantize_loss = (1.25 / (n * d)) * jnp.sum(parts)
    return (quantized_st, indices, quantize_loss)
